# Initial kernel scaffold; baseline (speedup 1.0000x reference)
import functools

import jax
import jax.numpy as jnp
from jax.experimental import pallas as pl


_K = 16


def _topk_moments_kernel(uvt_ref, uvq_ref, xw_ref, out_ref):
    R = uvq_ref.shape[2]
    Mloc = uvt_ref.shape[2]
    ux = uvt_ref[0, 0:1, :]
    uy = uvt_ref[0, 1:2, :]
    qx = uvq_ref[0, 0, :, 0:1]
    qy = uvq_ref[0, 0, :, 1:2]
    du = qx - ux
    dv = qy - uy
    d = du * du + dv * dv
    iota = jax.lax.broadcasted_iota(jnp.int32, (R, Mloc), 1)

    def body(t, carry):
        d, s = carry
        m = jnp.min(d, axis=1, keepdims=True)
        idx = jnp.min(jnp.where(d == m, iota, Mloc), axis=1, keepdims=True)
        onehot = iota == idx
        s = jnp.where(onehot, jnp.float32(1.0), s)
        d = jnp.where(onehot, jnp.float32(jnp.inf), d)
        return d, s

    s0 = jnp.zeros((R, Mloc), jnp.float32)
    _, s = jax.lax.fori_loop(0, _K, body, (d, s0))
    out_ref[0, :, :] = jnp.dot(s, xw_ref[0], preferred_element_type=jnp.float32)


def _eig_kernel(m_ref, out_ref):
    k = jnp.float32(_K)
    sx = m_ref[0:1, :]
    sy = m_ref[1:2, :]
    sz = m_ref[2:3, :]
    c00 = m_ref[3:4, :] - sx * sx / k
    c11 = m_ref[4:5, :] - sy * sy / k
    c22 = m_ref[5:6, :] - sz * sz / k
    c01 = m_ref[6:7, :] - sx * sy / k
    c02 = m_ref[7:8, :] - sx * sz / k
    c12 = m_ref[8:9, :] - sy * sz / k

    q = (c00 + c11 + c22) / 3.0
    b00 = c00 - q
    b11 = c11 - q
    b22 = c22 - q
    p1 = c01 * c01 + c02 * c02 + c12 * c12
    p2 = b00 * b00 + b11 * b11 + b22 * b22 + 2.0 * p1
    p = jnp.sqrt(p2 / 6.0)
    det = (b00 * (b11 * b22 - c12 * c12)
           - c01 * (c01 * b22 - c12 * c02)
           + c02 * (c01 * c12 - b11 * c02))
    p_safe = jnp.maximum(p, jnp.float32(1e-20))
    r = jnp.clip(det / (2.0 * p_safe * p_safe * p_safe), -1.0, 1.0)
    phi = jnp.arccos(r) / 3.0
    e1 = q + 2.0 * p * jnp.cos(phi)
    e3 = q + 2.0 * p * jnp.cos(phi + jnp.float32(2.0943951023931953))
    e2 = 3.0 * q - e1 - e3
    out_ref[0:1, :] = e1
    out_ref[1:2, :] = e2
    out_ref[2:3, :] = e3


@jax.jit
def kernel(X, uv):
    B, M, _ = X.shape
    R = 256
    uvt = jnp.transpose(uv, (0, 2, 1))
    uvq = uv.reshape(B, M // R, R, 2)
    x, y, z = X[..., 0], X[..., 1], X[..., 2]
    xw = jnp.stack([x, y, z, x * x, y * y, z * z, x * y, x * z, y * z],
                   axis=-1)

    mom = pl.pallas_call(
        _topk_moments_kernel,
        grid=(B, M // R),
        in_specs=[
            pl.BlockSpec((1, 2, M), lambda b, j: (b, 0, 0)),
            pl.BlockSpec((1, 1, R, 2), lambda b, j: (b, j, 0, 0)),
            pl.BlockSpec((1, M, 9), lambda b, j: (b, 0, 0)),
        ],
        out_specs=pl.BlockSpec((1, R, 9), lambda b, j: (b, j, 0)),
        out_shape=jax.ShapeDtypeStruct((B, M, 9), jnp.float32),
    )(uvt, uvq, xw)

    m9 = mom.reshape(B * M, 9).T
    eig = pl.pallas_call(
        _eig_kernel,
        out_shape=jax.ShapeDtypeStruct((3, B * M), jnp.float32),
    )(m9)
    return eig.T.reshape(B, M, 3)


# speedup vs baseline: 37.7950x; 37.7950x over previous
import functools

import jax
import jax.numpy as jnp
from jax.experimental import pallas as pl


_K = 16


def _topk_moments_kernel(uvt_ref, uvq_ref, xw_ref, out_ref):
    R = uvq_ref.shape[2]
    Mloc = uvt_ref.shape[2]
    ux = uvt_ref[0, 0:1, :]
    uy = uvt_ref[0, 1:2, :]
    qx = uvq_ref[0, 0, :, 0:1]
    qy = uvq_ref[0, 0, :, 1:2]
    du = qx - ux
    dv = qy - uy
    d = du * du + dv * dv
    iota = jax.lax.broadcasted_iota(jnp.int32, (R, Mloc), 1)

    def body(t, carry):
        d, s = carry
        m = jnp.min(d, axis=1, keepdims=True)
        idx = jnp.min(jnp.where(d == m, iota, Mloc), axis=1, keepdims=True)
        onehot = iota == idx
        s = jnp.where(onehot, jnp.float32(1.0), s)
        d = jnp.where(onehot, jnp.float32(jnp.inf), d)
        return d, s

    s0 = jnp.zeros((R, Mloc), jnp.float32)
    _, s = jax.lax.fori_loop(0, _K, body, (d, s0))
    out_ref[0, :, :] = jnp.dot(s, xw_ref[0], preferred_element_type=jnp.float32)


def _eig_kernel(m_ref, out_ref):
    k = jnp.float32(_K)
    sx = m_ref[0:1, :]
    sy = m_ref[1:2, :]
    sz = m_ref[2:3, :]
    c00 = m_ref[3:4, :] - sx * sx / k
    c11 = m_ref[4:5, :] - sy * sy / k
    c22 = m_ref[5:6, :] - sz * sz / k
    c01 = m_ref[6:7, :] - sx * sy / k
    c02 = m_ref[7:8, :] - sx * sz / k
    c12 = m_ref[8:9, :] - sy * sz / k

    one = jnp.float32(1.0)
    zero = jnp.float32(0.0)

    def rot(app, aqq, apq, arp, arq):
        denom = 2.0 * apq
        theta = (aqq - app) / jnp.where(denom == zero, one, denom)
        sgn = jnp.where(theta >= zero, one, -one)
        t = sgn / (jnp.abs(theta) + jnp.sqrt(theta * theta + one))
        t = jnp.where(apq == zero, zero, t)
        c = jax.lax.rsqrt(t * t + one)
        s = t * c
        napp = app - t * apq
        naqq = aqq + t * apq
        narp = c * arp - s * arq
        narq = s * arp + c * arq
        return napp, naqq, narp, narq

    a00, a11, a22, a01, a02, a12 = c00, c11, c22, c01, c02, c12
    for _ in range(6):
        a00, a11, a02, a12 = rot(a00, a11, a01, a02, a12)
        a01 = zero * a01
        a00, a22, a01, a12 = rot(a00, a22, a02, a01, a12)
        a02 = zero * a02
        a11, a22, a01, a02 = rot(a11, a22, a12, a01, a02)
        a12 = zero * a12

    e1 = jnp.maximum(jnp.maximum(a00, a11), a22)
    e3 = jnp.minimum(jnp.minimum(a00, a11), a22)
    e2 = (a00 + a11 + a22) - e1 - e3
    out_ref[0:1, :] = e1
    out_ref[1:2, :] = e2
    out_ref[2:3, :] = e3


@jax.jit
def kernel(X, uv):
    B, M, _ = X.shape
    R = 256
    uvt = jnp.transpose(uv, (0, 2, 1))
    uvq = uv.reshape(B, M // R, R, 2)
    x, y, z = X[..., 0], X[..., 1], X[..., 2]
    xw = jnp.stack([x, y, z, x * x, y * y, z * z, x * y, x * z, y * z],
                   axis=-1)

    mom = pl.pallas_call(
        _topk_moments_kernel,
        grid=(B, M // R),
        in_specs=[
            pl.BlockSpec((1, 2, M), lambda b, j: (b, 0, 0)),
            pl.BlockSpec((1, 1, R, 2), lambda b, j: (b, j, 0, 0)),
            pl.BlockSpec((1, M, 9), lambda b, j: (b, 0, 0)),
        ],
        out_specs=pl.BlockSpec((1, R, 9), lambda b, j: (b, j, 0)),
        out_shape=jax.ShapeDtypeStruct((B, M, 9), jnp.float32),
    )(uvt, uvq, xw)

    m9 = mom.reshape(B * M, 9).T
    eig = pl.pallas_call(
        _eig_kernel,
        out_shape=jax.ShapeDtypeStruct((3, B * M), jnp.float32),
    )(m9)
    return eig.T.reshape(B, M, 3)


# speedup vs baseline: 82.3126x; 2.1779x over previous
import functools

import jax
import jax.numpy as jnp
from jax import lax
from jax.experimental import pallas as pl
from jax.experimental.pallas import tpu as pltpu
from jax.experimental.pallas import tpu_sc as plsc


_K = 16
_NC = 2
_NS = 16
_NW = _NC * _NS


def _sc_stage1(ux, uy, x, y, z, B, M):
    N = B * M
    ppt = N // _NW
    seg = M // ppt
    nchunk = M // 16

    @functools.partial(
        pl.kernel,
        mesh=plsc.VectorSubcoreMesh(core_axis_name="c", subcore_axis_name="s"),
        out_type=jax.ShapeDtypeStruct((N, 16), jnp.float32),
        compiler_params=pltpu.CompilerParams(needs_layout_passes=False),
        scratch_types=[
            pltpu.VMEM((M,), jnp.float32),
            pltpu.VMEM((M,), jnp.float32),
            pltpu.VMEM((M,), jnp.float32),
            pltpu.VMEM((M,), jnp.float32),
            pltpu.VMEM((M,), jnp.float32),
            pltpu.VMEM((ppt, 16), jnp.float32),
        ],
    )
    def body(ux_hbm, uy_hbm, x_hbm, y_hbm, z_hbm, out_hbm,
             ux_v, uy_v, x_v, y_v, z_v, mom_v):
        cid = lax.axis_index("c")
        sid = lax.axis_index("s")
        wid = sid * _NC + cid
        base = wid * ppt
        boff = (wid // seg) * M
        pltpu.sync_copy(ux_hbm.at[pl.ds(boff, M)], ux_v)
        pltpu.sync_copy(uy_hbm.at[pl.ds(boff, M)], uy_v)
        pltpu.sync_copy(x_hbm.at[pl.ds(boff, M)], x_v)
        pltpu.sync_copy(y_hbm.at[pl.ds(boff, M)], y_v)
        pltpu.sync_copy(z_hbm.at[pl.ds(boff, M)], z_v)

        lane = lax.iota(jnp.int32, 16)
        zero16 = jnp.zeros((16,), jnp.int32)
        inf16 = jnp.full((16,), jnp.float32(jnp.inf))

        def point_body(p, _):
            loc16 = jnp.full((16,), (wid % seg) * ppt + p, jnp.int32)
            qx = plsc.load_gather(ux_v, [loc16])
            qy = plsc.load_gather(uy_v, [loc16])

            def chunk_body(ci, carry):
                bd, bi = carry
                off = pl.multiple_of(ci * 16, 16)
                cx = ux_v[pl.ds(off, 16)]
                cy = uy_v[pl.ds(off, 16)]
                du = cx - qx
                dv = cy - qy
                d = du * du + dv * dv
                sd, si = plsc.sort_key_val(d, lane + off)
                rd = lax.rev(sd, (0,))
                ri = lax.rev(si, (0,))
                take = rd < bd
                nd = jnp.where(take, rd, bd)
                ni = jnp.where(take, ri, bi)
                nbd, nbi = plsc.sort_key_val(nd, ni)
                return (nbd, nbi)

            _, bi = lax.fori_loop(0, nchunk, chunk_body, (inf16, zero16))
            gx = plsc.load_gather(x_v, [bi])
            gy = plsc.load_gather(y_v, [bi])
            gz = plsc.load_gather(z_v, [bi])
            sums = (gx, gy, gz, gx * gx, gy * gy, gz * gz,
                    gx * gy, gx * gz, gy * gz)
            mom = jnp.zeros((16,), jnp.float32)
            for j, v in enumerate(sums):
                mom = mom + jnp.where(lane == j, jnp.sum(v), jnp.float32(0.0))
            mom_v[p] = mom
            return 0

        lax.fori_loop(0, ppt, point_body, 0)
        pltpu.sync_copy(mom_v, out_hbm.at[pl.ds(base, ppt)])

    return body(ux, uy, x, y, z)


def _eig_kernel(m_ref, out_ref):
    k = jnp.float32(_K)
    sx = m_ref[0:1, :]
    sy = m_ref[1:2, :]
    sz = m_ref[2:3, :]
    c00 = m_ref[3:4, :] - sx * sx / k
    c11 = m_ref[4:5, :] - sy * sy / k
    c22 = m_ref[5:6, :] - sz * sz / k
    c01 = m_ref[6:7, :] - sx * sy / k
    c02 = m_ref[7:8, :] - sx * sz / k
    c12 = m_ref[8:9, :] - sy * sz / k

    one = jnp.float32(1.0)
    zero = jnp.float32(0.0)

    def rot(app, aqq, apq, arp, arq):
        denom = 2.0 * apq
        theta = (aqq - app) / jnp.where(denom == zero, one, denom)
        sgn = jnp.where(theta >= zero, one, -one)
        t = sgn / (jnp.abs(theta) + jnp.sqrt(theta * theta + one))
        t = jnp.where(apq == zero, zero, t)
        c = lax.rsqrt(t * t + one)
        s = t * c
        napp = app - t * apq
        naqq = aqq + t * apq
        narp = c * arp - s * arq
        narq = s * arp + c * arq
        return napp, naqq, narp, narq

    a00, a11, a22, a01, a02, a12 = c00, c11, c22, c01, c02, c12
    for _ in range(6):
        a00, a11, a02, a12 = rot(a00, a11, a01, a02, a12)
        a01 = zero * a01
        a00, a22, a01, a12 = rot(a00, a22, a02, a01, a12)
        a02 = zero * a02
        a11, a22, a01, a02 = rot(a11, a22, a12, a01, a02)
        a12 = zero * a12

    e1 = jnp.maximum(jnp.maximum(a00, a11), a22)
    e3 = jnp.minimum(jnp.minimum(a00, a11), a22)
    e2 = (a00 + a11 + a22) - e1 - e3
    out_ref[0:1, :] = e1
    out_ref[1:2, :] = e2
    out_ref[2:3, :] = e3


@jax.jit
def kernel(X, uv):
    B, M, _ = X.shape
    mom16 = _sc_stage1(uv[..., 0].reshape(-1), uv[..., 1].reshape(-1),
                       X[..., 0].reshape(-1), X[..., 1].reshape(-1),
                       X[..., 2].reshape(-1), B, M)
    m9 = mom16[:, :9].T
    eig = pl.pallas_call(
        _eig_kernel,
        out_shape=jax.ShapeDtypeStruct((3, B * M), jnp.float32),
    )(m9)
    return eig.T.reshape(B, M, 3)


# speedup vs baseline: 242.3795x; 2.9446x over previous
import functools

import jax
import jax.numpy as jnp
from jax import lax
from jax.experimental import pallas as pl
from jax.experimental.pallas import tpu as pltpu
from jax.experimental.pallas import tpu_sc as plsc


_K = 16
_NC = 2
_NS = 16
_NW = _NC * _NS


def _sc_stage1(ux, uy, x, y, z, B, M):
    N = B * M
    ppt = N // _NW
    seg = M // ppt
    nchunk = M // 16

    @functools.partial(
        pl.kernel,
        mesh=plsc.VectorSubcoreMesh(core_axis_name="c", subcore_axis_name="s"),
        out_type=jax.ShapeDtypeStruct((N, 16), jnp.float32),
        compiler_params=pltpu.CompilerParams(needs_layout_passes=False),
        scratch_types=[
            pltpu.VMEM((M,), jnp.float32),
            pltpu.VMEM((M,), jnp.float32),
            pltpu.VMEM((M,), jnp.float32),
            pltpu.VMEM((M,), jnp.float32),
            pltpu.VMEM((M,), jnp.float32),
            pltpu.VMEM((ppt, 16), jnp.float32),
        ],
    )
    def body(ux_hbm, uy_hbm, x_hbm, y_hbm, z_hbm, out_hbm,
             ux_v, uy_v, x_v, y_v, z_v, mom_v):
        cid = lax.axis_index("c")
        sid = lax.axis_index("s")
        wid = sid * _NC + cid
        base = wid * ppt
        boff = (wid // seg) * M
        pltpu.sync_copy(ux_hbm.at[pl.ds(boff, M)], ux_v)
        pltpu.sync_copy(uy_hbm.at[pl.ds(boff, M)], uy_v)
        pltpu.sync_copy(x_hbm.at[pl.ds(boff, M)], x_v)
        pltpu.sync_copy(y_hbm.at[pl.ds(boff, M)], y_v)
        pltpu.sync_copy(z_hbm.at[pl.ds(boff, M)], z_v)

        lane = lax.iota(jnp.int32, 16)
        zero16 = jnp.zeros((16,), jnp.int32)
        inf16 = jnp.full((16,), jnp.float32(jnp.inf))

        def point_body(p, _):
            loc16 = jnp.full((16,), (wid % seg) * ppt + p, jnp.int32)
            qx = plsc.load_gather(ux_v, [loc16])
            qy = plsc.load_gather(uy_v, [loc16])

            def merge(a, b):
                rd = lax.rev(b[0], (0,))
                ri = lax.rev(b[1], (0,))
                take = rd < a[0]
                nd = jnp.where(take, rd, a[0])
                ni = jnp.where(take, ri, a[1])
                nk, nv = plsc.sort_key_val(nd, ni)
                return (nk, nv)

            def chunk_body(ci, carry):
                off0 = pl.multiple_of(ci * 128, 128)
                level = []
                for j in range(8):
                    off = off0 + j * 16
                    cx = ux_v[pl.ds(off, 16)]
                    cy = uy_v[pl.ds(off, 16)]
                    du = cx - qx
                    dv = cy - qy
                    d = du * du + dv * dv
                    sd, si = plsc.sort_key_val(d, lane + off)
                    level.append((sd, si))
                while len(level) > 1:
                    level = [merge(level[i], level[i + 1])
                             for i in range(0, len(level), 2)]
                return merge(carry, level[0])

            _, bi = lax.fori_loop(0, nchunk // 8, chunk_body, (inf16, zero16))
            gx = plsc.load_gather(x_v, [bi])
            gy = plsc.load_gather(y_v, [bi])
            gz = plsc.load_gather(z_v, [bi])
            sums = (gx, gy, gz, gx * gx, gy * gy, gz * gz,
                    gx * gy, gx * gz, gy * gz)
            mom = jnp.zeros((16,), jnp.float32)
            for j, v in enumerate(sums):
                mom = mom + jnp.where(lane == j, jnp.sum(v), jnp.float32(0.0))
            mom_v[p] = mom
            return 0

        lax.fori_loop(0, ppt, point_body, 0)
        pltpu.sync_copy(mom_v, out_hbm.at[pl.ds(base, ppt)])

    return body(ux, uy, x, y, z)


def _eig_kernel(m_ref, out_ref):
    k = jnp.float32(_K)
    sx = m_ref[0:1, :]
    sy = m_ref[1:2, :]
    sz = m_ref[2:3, :]
    c00 = m_ref[3:4, :] - sx * sx / k
    c11 = m_ref[4:5, :] - sy * sy / k
    c22 = m_ref[5:6, :] - sz * sz / k
    c01 = m_ref[6:7, :] - sx * sy / k
    c02 = m_ref[7:8, :] - sx * sz / k
    c12 = m_ref[8:9, :] - sy * sz / k

    one = jnp.float32(1.0)
    zero = jnp.float32(0.0)

    def rot(app, aqq, apq, arp, arq):
        denom = 2.0 * apq
        theta = (aqq - app) / jnp.where(denom == zero, one, denom)
        sgn = jnp.where(theta >= zero, one, -one)
        t = sgn / (jnp.abs(theta) + jnp.sqrt(theta * theta + one))
        t = jnp.where(apq == zero, zero, t)
        c = lax.rsqrt(t * t + one)
        s = t * c
        napp = app - t * apq
        naqq = aqq + t * apq
        narp = c * arp - s * arq
        narq = s * arp + c * arq
        return napp, naqq, narp, narq

    a00, a11, a22, a01, a02, a12 = c00, c11, c22, c01, c02, c12
    for _ in range(6):
        a00, a11, a02, a12 = rot(a00, a11, a01, a02, a12)
        a01 = zero * a01
        a00, a22, a01, a12 = rot(a00, a22, a02, a01, a12)
        a02 = zero * a02
        a11, a22, a01, a02 = rot(a11, a22, a12, a01, a02)
        a12 = zero * a12

    e1 = jnp.maximum(jnp.maximum(a00, a11), a22)
    e3 = jnp.minimum(jnp.minimum(a00, a11), a22)
    e2 = (a00 + a11 + a22) - e1 - e3
    out_ref[0:1, :] = e1
    out_ref[1:2, :] = e2
    out_ref[2:3, :] = e3


@jax.jit
def kernel(X, uv):
    B, M, _ = X.shape
    mom16 = _sc_stage1(uv[..., 0].reshape(-1), uv[..., 1].reshape(-1),
                       X[..., 0].reshape(-1), X[..., 1].reshape(-1),
                       X[..., 2].reshape(-1), B, M)
    m9 = mom16[:, :9].T
    eig = pl.pallas_call(
        _eig_kernel,
        out_shape=jax.ShapeDtypeStruct((3, B * M), jnp.float32),
    )(m9)
    return eig.T.reshape(B, M, 3)


# speedup vs baseline: 260.0805x; 1.0730x over previous
import functools

import jax
import jax.numpy as jnp
from jax import lax
from jax.experimental import pallas as pl
from jax.experimental.pallas import tpu as pltpu
from jax.experimental.pallas import tpu_sc as plsc


_K = 16
_NC = 2
_NS = 16
_NW = _NC * _NS


def _sc_stage1(ux, uy, x, y, z, B, M):
    N = B * M
    ppt = N // _NW
    seg = M // ppt
    nchunk = M // 16

    @functools.partial(
        pl.kernel,
        mesh=plsc.VectorSubcoreMesh(core_axis_name="c", subcore_axis_name="s"),
        out_type=jax.ShapeDtypeStruct((N, 16), jnp.float32),
        compiler_params=pltpu.CompilerParams(needs_layout_passes=False),
        scratch_types=[
            pltpu.VMEM((M,), jnp.float32),
            pltpu.VMEM((M,), jnp.float32),
            pltpu.VMEM((M,), jnp.float32),
            pltpu.VMEM((M,), jnp.float32),
            pltpu.VMEM((M,), jnp.float32),
            pltpu.VMEM((ppt, 16), jnp.float32),
        ],
    )
    def body(ux_hbm, uy_hbm, x_hbm, y_hbm, z_hbm, out_hbm,
             ux_v, uy_v, x_v, y_v, z_v, mom_v):
        cid = lax.axis_index("c")
        sid = lax.axis_index("s")
        wid = sid * _NC + cid
        base = wid * ppt
        boff = (wid // seg) * M
        pltpu.sync_copy(ux_hbm.at[pl.ds(boff, M)], ux_v)
        pltpu.sync_copy(uy_hbm.at[pl.ds(boff, M)], uy_v)
        pltpu.sync_copy(x_hbm.at[pl.ds(boff, M)], x_v)
        pltpu.sync_copy(y_hbm.at[pl.ds(boff, M)], y_v)
        pltpu.sync_copy(z_hbm.at[pl.ds(boff, M)], z_v)

        lane = lax.iota(jnp.int32, 16)
        zero16 = jnp.zeros((16,), jnp.int32)
        inf16 = jnp.full((16,), jnp.float32(jnp.inf))

        def point_body(p, _):
            loc16 = jnp.full((16,), (wid % seg) * ppt + p, jnp.int32)
            qx = plsc.load_gather(ux_v, [loc16])
            qy = plsc.load_gather(uy_v, [loc16])

            def merge(a, b, descending):
                take = b[0] < a[0]
                nd = jnp.where(take, b[0], a[0])
                ni = jnp.where(take, b[1], a[1])
                nk, nv = plsc.sort_key_val(nd, ni, descending=descending)
                return (nk, nv)

            def chunk_body(ci, carry):
                off0 = pl.multiple_of(ci * 256, 256)
                level = []
                for j in range(16):
                    off = off0 + j * 16
                    cx = ux_v[pl.ds(off, 16)]
                    cy = uy_v[pl.ds(off, 16)]
                    du = cx - qx
                    dv = cy - qy
                    d = du * du + dv * dv
                    sd, si = plsc.sort_key_val(d, lane + off,
                                               descending=(j % 2 == 1))
                    level.append((sd, si))
                while len(level) > 2:
                    level = [merge(level[i], level[i + 1],
                                   descending=(i % 4 == 2))
                             for i in range(0, len(level), 2)]
                root = merge(level[0], level[1], descending=True)
                return merge(carry, root, descending=False)

            _, bi = lax.fori_loop(0, nchunk // 16, chunk_body,
                                  (inf16, zero16))
            gx = plsc.load_gather(x_v, [bi])
            gy = plsc.load_gather(y_v, [bi])
            gz = plsc.load_gather(z_v, [bi])
            sums = (gx, gy, gz, gx * gx, gy * gy, gz * gz,
                    gx * gy, gx * gz, gy * gz)
            mom = jnp.zeros((16,), jnp.float32)
            for j, v in enumerate(sums):
                mom = mom + jnp.where(lane == j, jnp.sum(v), jnp.float32(0.0))
            mom_v[p] = mom
            return 0

        lax.fori_loop(0, ppt, point_body, 0)
        pltpu.sync_copy(mom_v, out_hbm.at[pl.ds(base, ppt)])

    return body(ux, uy, x, y, z)


def _eig_kernel(m_ref, out_ref):
    k = jnp.float32(_K)
    sx = m_ref[0:1, :]
    sy = m_ref[1:2, :]
    sz = m_ref[2:3, :]
    c00 = m_ref[3:4, :] - sx * sx / k
    c11 = m_ref[4:5, :] - sy * sy / k
    c22 = m_ref[5:6, :] - sz * sz / k
    c01 = m_ref[6:7, :] - sx * sy / k
    c02 = m_ref[7:8, :] - sx * sz / k
    c12 = m_ref[8:9, :] - sy * sz / k

    one = jnp.float32(1.0)
    zero = jnp.float32(0.0)

    def rot(app, aqq, apq, arp, arq):
        denom = 2.0 * apq
        theta = (aqq - app) / jnp.where(denom == zero, one, denom)
        sgn = jnp.where(theta >= zero, one, -one)
        t = sgn / (jnp.abs(theta) + jnp.sqrt(theta * theta + one))
        t = jnp.where(apq == zero, zero, t)
        c = lax.rsqrt(t * t + one)
        s = t * c
        napp = app - t * apq
        naqq = aqq + t * apq
        narp = c * arp - s * arq
        narq = s * arp + c * arq
        return napp, naqq, narp, narq

    a00, a11, a22, a01, a02, a12 = c00, c11, c22, c01, c02, c12
    for _ in range(6):
        a00, a11, a02, a12 = rot(a00, a11, a01, a02, a12)
        a01 = zero * a01
        a00, a22, a01, a12 = rot(a00, a22, a02, a01, a12)
        a02 = zero * a02
        a11, a22, a01, a02 = rot(a11, a22, a12, a01, a02)
        a12 = zero * a12

    e1 = jnp.maximum(jnp.maximum(a00, a11), a22)
    e3 = jnp.minimum(jnp.minimum(a00, a11), a22)
    e2 = (a00 + a11 + a22) - e1 - e3
    out_ref[0:1, :] = e1
    out_ref[1:2, :] = e2
    out_ref[2:3, :] = e3


@jax.jit
def kernel(X, uv):
    B, M, _ = X.shape
    mom16 = _sc_stage1(uv[..., 0].reshape(-1), uv[..., 1].reshape(-1),
                       X[..., 0].reshape(-1), X[..., 1].reshape(-1),
                       X[..., 2].reshape(-1), B, M)
    m9 = mom16[:, :9].T
    eig = pl.pallas_call(
        _eig_kernel,
        out_shape=jax.ShapeDtypeStruct((3, B * M), jnp.float32),
    )(m9)
    return eig.T.reshape(B, M, 3)
